# baseline (speedup 1.0000x reference)
import functools

import jax
import jax.numpy as jnp
from jax import lax
from jax.experimental import pallas as pl
from jax.experimental.pallas import tpu as pltpu
from jax.experimental.pallas import tpu_sc as plsc


def _make_lookup(n, d, dtype):
    info = plsc.get_sparse_core_info()
    nc, ns = info.num_cores, info.num_subcores
    nw = nc * ns
    rows_per_w = n // nw
    rc = 32
    nchunks = rows_per_w // rc
    mesh = plsc.VectorSubcoreMesh(core_axis_name="c", subcore_axis_name="s")

    @functools.partial(
        pl.kernel,
        mesh=mesh,
        out_type=jax.ShapeDtypeStruct((n, d), dtype),
        scratch_types=[
            pltpu.VMEM((rc, d), dtype),
            pltpu.VMEM((rc, d), dtype),
            pltpu.SemaphoreType.DMA,
            pltpu.SemaphoreType.DMA,
            pltpu.SemaphoreType.DMA,
            pltpu.SemaphoreType.DMA,
        ],
    )
    def lookup(table_hbm, out_hbm, buf0, buf1, isem0, isem1, osem0, osem1):
        wid = lax.axis_index("s") * nc + lax.axis_index("c")
        base = wid * rows_per_w
        bufs = (buf0, buf1)
        isems = (isem0, isem1)
        osems = (osem0, osem1)

        cin = [None] * nchunks
        cout = [None] * nchunks
        cin[0] = pltpu.async_copy(table_hbm.at[pl.ds(base, rc)], buf0, isem0)
        for i in range(nchunks):
            b = i % 2
            cin[i].wait()
            cout[i] = pltpu.async_copy(
                bufs[b], out_hbm.at[pl.ds(base + i * rc, rc)], osems[b]
            )
            if i + 1 < nchunks:
                if i >= 1:
                    cout[i - 1].wait()
                nb = (i + 1) % 2
                cin[i + 1] = pltpu.async_copy(
                    table_hbm.at[pl.ds(base + (i + 1) * rc, rc)], bufs[nb], isems[nb]
                )
        cout[nchunks - 1].wait()

    return lookup


def kernel(seq_length, table):
    n, d = table.shape
    out = _make_lookup(n, d, table.dtype)(table)
    return out.reshape(1, n, d)
